# Initial kernel scaffold; baseline (speedup 1.0000x reference)
import jax
import jax.numpy as jnp
from jax.experimental import pallas as pl


def kernel(x, W1, b1, W2, b2):
    raise NotImplementedError("write your pallas kernel here")


# speedup vs baseline: 1.0530x; 1.0530x over previous
import jax
import jax.numpy as jnp
from jax.experimental import pallas as pl
from jax.experimental.pallas import tpu as pltpu


def _body(n_blocks, x_ref, w1_ref, b1_ref, w2_ref, b2_ref, out_ref):
    n = pl.program_id(1)
    xb = x_ref[...].astype(jnp.bfloat16)
    h = jnp.dot(xb, w1_ref[...], preferred_element_type=jnp.float32)
    h = jnp.maximum(h + b1_ref[...], 0.0).astype(jnp.bfloat16)
    p = jnp.dot(h, w2_ref[...], preferred_element_type=jnp.float32)

    @pl.when(n == 0)
    def _():
        out_ref[...] = p + b2_ref[...]

    @pl.when(n != 0)
    def _():
        out_ref[...] += p

    @pl.when(n == n_blocks - 1)
    def _():
        out_ref[...] = jax.nn.sigmoid(out_ref[...])


def _fused_mlp(x, W1, b1, W2, b2, bm, bn):
    m, k = x.shape
    n = W1.shape[1]
    o = W2.shape[1]
    bm = min(bm, m)
    bn = min(bn, n)
    n_blocks = n // bn
    import functools
    body = functools.partial(_body, n_blocks)
    return pl.pallas_call(
        body,
        grid=(m // bm, n_blocks),
        in_specs=[
            pl.BlockSpec((bm, k), lambda i, j: (i, 0)),
            pl.BlockSpec((k, bn), lambda i, j: (0, j)),
            pl.BlockSpec((1, bn), lambda i, j: (0, j)),
            pl.BlockSpec((bn, o), lambda i, j: (j, 0)),
            pl.BlockSpec((1, o), lambda i, j: (0, 0)),
        ],
        out_specs=pl.BlockSpec((bm, o), lambda i, j: (i, 0)),
        out_shape=jax.ShapeDtypeStruct((m, o), jnp.float32),
        compiler_params=pltpu.CompilerParams(
            dimension_semantics=("parallel", "arbitrary"),
        ),
    )(x, W1.astype(jnp.bfloat16), b1.reshape(1, n),
      W2.astype(jnp.bfloat16), b2.reshape(1, o))


def kernel(x, W1, b1, W2, b2):
    return _fused_mlp(x, W1, b1, W2, b2, bm=1024, bn=1024)
